# Initial kernel scaffold; baseline (speedup 1.0000x reference)
import jax
import jax.numpy as jnp
from jax.experimental import pallas as pl


def kernel(tokens_f, tokens_g, embedding, state_zero):
    raise NotImplementedError("write your pallas kernel here")


# speedup vs baseline: 3.5424x; 3.5424x over previous
import functools

import jax
import jax.numpy as jnp
from jax import lax
from jax.experimental import pallas as pl
from jax.experimental.pallas import tpu as pltpu
from jax.experimental.pallas import tpu_sc as plsc

B = 4096
S = 200
D = 32
L = 16
NC = 2
NS = 16
NW = NC * NS
RPW = B // NW
C0 = 128
C1 = S - C0


def _accum(buf, z0, z1):
    v0a = buf[0, pl.ds(0, L)]
    v0b = buf[0, pl.ds(L, L)]
    zero = jnp.zeros((L,), jnp.float32)

    def step(s, carry):
        sa, sb, qa, qb, da, db, pa, pb = carry
        va = buf[s, pl.ds(0, L)]
        vb = buf[s, pl.ds(L, L)]
        ta = va - pa
        tb = vb - pb
        return (sa + va, sb + vb, qa + va * va, qb + vb * vb,
                da + ta * ta, db + tb * tb, va, vb)

    sa, sb, qa, qb, da, db, pa, pb = lax.fori_loop(
        0, S, step, (zero, zero, zero, zero, zero, zero, v0a, v0b), unroll=8)
    inv_s = jnp.float32(1.0 / S)
    inv_d = jnp.float32(1.0 / (S - 1))
    ma = sa * inv_s - z0
    mb = sb * inv_s - z1
    ea = qa * inv_s - 2.0 * z0 * (sa * inv_s) + z0 * z0
    eb = qb * inv_s - 2.0 * z1 * (sb * inv_s) + z1 * z1
    dma = (pa - v0a) * inv_d
    dmb = (pb - v0b) * inv_d
    dea = da * inv_d
    deb = db * inv_d
    return (ma, mb, ea, eb, dma, dmb, dea, deb)


def _sc_body(tf_hbm, tg_hbm, emb_hbm, z_hbm, out_hbm,
             tf_v, tg_v, z_v, buf_f, buf_g, out_v, sem_f, sem_g):
    cid = lax.axis_index("c")
    sid = lax.axis_index("s")
    wid = sid * NC + cid
    base = wid * RPW

    pltpu.sync_copy(tf_hbm.at[pl.ds(base, RPW), :], tf_v)
    pltpu.sync_copy(tg_hbm.at[pl.ds(base, RPW), :], tg_v)
    pltpu.sync_copy(z_hbm, z_v)
    z0 = z_v[pl.ds(0, L)]
    z1 = z_v[pl.ds(L, L)]

    iota = lax.iota(jnp.int32, L)
    lane0 = iota == 0
    bfly = [jnp.bitwise_xor(iota, k) for k in (8, 4, 2, 1)]

    def row_body(r, carry):
        df0 = pltpu.async_copy(
            emb_hbm.at[tf_v.at[r, pl.ds(0, C0)]], buf_f.at[pl.ds(0, C0), :], sem_f)
        df1 = pltpu.async_copy(
            emb_hbm.at[tf_v.at[r, pl.ds(C0, C1)]], buf_f.at[pl.ds(C0, C1), :], sem_f)
        dg0 = pltpu.async_copy(
            emb_hbm.at[tg_v.at[r, pl.ds(0, C0)]], buf_g.at[pl.ds(0, C0), :], sem_g)
        dg1 = pltpu.async_copy(
            emb_hbm.at[tg_v.at[r, pl.ds(C0, C1)]], buf_g.at[pl.ds(C0, C1), :], sem_g)
        df0.wait()
        df1.wait()
        sig_f = _accum(buf_f, z0, z1)
        dg0.wait()
        dg1.wait()
        sig_g = _accum(buf_g, z0, z1)
        acc = jnp.zeros((L,), jnp.float32)
        for f, g in zip(sig_f, sig_g):
            d = f - g
            acc = acc + d * d
        for idx in bfly:
            acc = acc + acc.at[idx].get(mode="promise_in_bounds")
        dist = acc * jnp.float32(1.0 / (4 * D))
        plsc.store_scatter(out_v, [jnp.full((L,), r, jnp.int32)], dist, mask=lane0)
        return carry

    lax.fori_loop(0, RPW, row_body, 0)
    pltpu.sync_copy(out_v, out_hbm.at[pl.ds(base, RPW)])


def kernel(tokens_f, tokens_g, embedding, state_zero):
    mesh = plsc.VectorSubcoreMesh(
        core_axis_name="c", subcore_axis_name="s", num_cores=NC, num_subcores=NS)
    run = pl.kernel(
        _sc_body,
        out_type=jax.ShapeDtypeStruct((B,), jnp.float32),
        mesh=mesh,
        compiler_params=pltpu.CompilerParams(
            needs_layout_passes=False, use_tc_tiling_on_sc=False),
        scratch_types=[
            pltpu.VMEM((RPW, S), jnp.int32),
            pltpu.VMEM((RPW, S), jnp.int32),
            pltpu.VMEM((D,), jnp.float32),
            pltpu.VMEM((S, D), jnp.float32),
            pltpu.VMEM((S, D), jnp.float32),
            pltpu.VMEM((RPW,), jnp.float32),
            pltpu.SemaphoreType.DMA,
            pltpu.SemaphoreType.DMA,
        ],
    )
    return run(tokens_f.astype(jnp.int32), tokens_g.astype(jnp.int32),
               embedding, state_zero)


# speedup vs baseline: 4.2273x; 1.1933x over previous
import jax
import jax.numpy as jnp
from jax import lax
from jax.experimental import pallas as pl
from jax.experimental.pallas import tpu as pltpu
from jax.experimental.pallas import tpu_sc as plsc

B = 4096
S = 200
D = 32
L = 16
NC = 2
NS = 16
NW = NC * NS
RPW = B // NW
C0 = 128
C1 = S - C0
INV_S = 1.0 / S
INV_D = 1.0 / (S - 1)


def _sc_body(tf_hbm, tg_hbm, emb_hbm, z_hbm, out_hbm,
             tf_v, tg_v, z_v, buf_f0, buf_g0, buf_f1, buf_g1, out_v,
             sem0, sem1):
    cid = lax.axis_index("c")
    sid = lax.axis_index("s")
    wid = sid * NC + cid
    base = wid * RPW

    pltpu.sync_copy(tf_hbm.at[pl.ds(base, RPW), :], tf_v)
    pltpu.sync_copy(tg_hbm.at[pl.ds(base, RPW), :], tg_v)
    pltpu.sync_copy(z_hbm, z_v)
    z0 = z_v[pl.ds(0, L)]
    z1 = z_v[pl.ds(L, L)]

    iota = lax.iota(jnp.int32, L)
    lane0 = iota == 0
    bfly = [jnp.bitwise_xor(iota, k) for k in (8, 4, 2, 1)]

    def issue_pair(r, buf_f, buf_g, sem):
        pltpu.async_copy(
            emb_hbm.at[tf_v.at[r, pl.ds(0, C0)]], buf_f.at[pl.ds(0, C0), :], sem)
        pltpu.async_copy(
            emb_hbm.at[tf_v.at[r, pl.ds(C0, C1)]], buf_f.at[pl.ds(C0, C1), :], sem)
        pltpu.async_copy(
            emb_hbm.at[tg_v.at[r, pl.ds(0, C0)]], buf_g.at[pl.ds(0, C0), :], sem)
        pltpu.async_copy(
            emb_hbm.at[tg_v.at[r, pl.ds(C0, C1)]], buf_g.at[pl.ds(C0, C1), :], sem)

    def drain_pair(buf_f, buf_g, sem):
        pltpu.make_async_copy(emb_hbm.at[pl.ds(0, S), :], buf_f, sem).wait()
        pltpu.make_async_copy(emb_hbm.at[pl.ds(0, S), :], buf_g, sem).wait()

    def compute_row(r, buf_f, buf_g):
        f0a = buf_f[0, pl.ds(0, L)]
        f0b = buf_f[0, pl.ds(L, L)]
        g0a = buf_g[0, pl.ds(0, L)]
        g0b = buf_g[0, pl.ds(L, L)]
        zv = jnp.zeros((L,), jnp.float32)

        def step(s, carry):
            (fsa, fsb, fqa, fqb, fca, fcb, fpa, fpb,
             gsa, gsb, gqa, gqb, gca, gcb, gpa, gpb) = carry
            fva = buf_f[s, pl.ds(0, L)]
            fvb = buf_f[s, pl.ds(L, L)]
            gva = buf_g[s, pl.ds(0, L)]
            gvb = buf_g[s, pl.ds(L, L)]
            return (fsa + fva, fsb + fvb, fqa + fva * fva, fqb + fvb * fvb,
                    fca + fva * fpa, fcb + fvb * fpb, fva, fvb,
                    gsa + gva, gsb + gvb, gqa + gva * gva, gqb + gvb * gvb,
                    gca + gva * gpa, gcb + gvb * gpb, gva, gvb)

        (fsa, fsb, fqa, fqb, fca, fcb, fpa, fpb,
         gsa, gsb, gqa, gqb, gca, gcb, gpa, gpb) = lax.fori_loop(
            0, S, step,
            (zv, zv, zv, zv, zv, zv, f0a, f0b,
             zv, zv, zv, zv, zv, zv, g0a, g0b), unroll=8)

        inv_s = jnp.float32(INV_S)
        inv_d = jnp.float32(INV_D)
        acc = jnp.zeros((L,), jnp.float32)
        for (fs, fq, fc, fp, f0, gs, gq, gc, gp, g0, z) in (
                (fsa, fqa, fca, fpa, f0a, gsa, gqa, gca, gpa, g0a, z0),
                (fsb, fqb, fcb, fpb, f0b, gsb, gqb, gcb, gpb, g0b, z1)):
            sd = (fs - gs) * inv_s
            ed = (fq - gq) * inv_s - 2.0 * z * sd
            dmd = ((fp - f0) - (gp - g0)) * inv_d
            fdq = 2.0 * fq + f0 * f0 - fp * fp - 2.0 * fc
            gdq = 2.0 * gq + g0 * g0 - gp * gp - 2.0 * gc
            ded = (fdq - gdq) * inv_d
            acc = acc + sd * sd + ed * ed + dmd * dmd + ded * ded
        for idx in bfly:
            acc = acc + acc.at[idx].get(mode="promise_in_bounds")
        dist = acc * jnp.float32(1.0 / (4 * D))
        plsc.store_scatter(out_v, [jnp.full((L,), r, jnp.int32)], dist, mask=lane0)

    issue_pair(0, buf_f0, buf_g0, sem0)

    def pair_body(rr, carry):
        r0 = 2 * rr
        issue_pair(r0 + 1, buf_f1, buf_g1, sem1)
        drain_pair(buf_f0, buf_g0, sem0)
        compute_row(r0, buf_f0, buf_g0)

        @pl.when(rr < RPW // 2 - 1)
        def _():
            issue_pair(r0 + 2, buf_f0, buf_g0, sem0)

        drain_pair(buf_f1, buf_g1, sem1)
        compute_row(r0 + 1, buf_f1, buf_g1)
        return carry

    lax.fori_loop(0, RPW // 2, pair_body, 0)
    pltpu.sync_copy(out_v, out_hbm.at[pl.ds(base, RPW)])


def kernel(tokens_f, tokens_g, embedding, state_zero):
    mesh = plsc.VectorSubcoreMesh(
        core_axis_name="c", subcore_axis_name="s", num_cores=NC, num_subcores=NS)
    run = pl.kernel(
        _sc_body,
        out_type=jax.ShapeDtypeStruct((B,), jnp.float32),
        mesh=mesh,
        compiler_params=pltpu.CompilerParams(
            needs_layout_passes=False, use_tc_tiling_on_sc=False),
        scratch_types=[
            pltpu.VMEM((RPW, S), jnp.int32),
            pltpu.VMEM((RPW, S), jnp.int32),
            pltpu.VMEM((D,), jnp.float32),
            pltpu.VMEM((S, D), jnp.float32),
            pltpu.VMEM((S, D), jnp.float32),
            pltpu.VMEM((S, D), jnp.float32),
            pltpu.VMEM((S, D), jnp.float32),
            pltpu.VMEM((RPW,), jnp.float32),
            pltpu.SemaphoreType.DMA,
            pltpu.SemaphoreType.DMA,
        ],
    )
    return run(tokens_f.astype(jnp.int32), tokens_g.astype(jnp.int32),
               embedding, state_zero)
